# Initial kernel scaffold; baseline (speedup 1.0000x reference)
import jax
import jax.numpy as jnp
from jax.experimental import pallas as pl


def kernel(predict_pc, gt_pc):
    raise NotImplementedError("write your pallas kernel here")


# speedup vs baseline: 2.9519x; 2.9519x over previous
import jax
import jax.numpy as jnp
from jax.experimental import pallas as pl
from jax.experimental.pallas import tpu as pltpu

B = 8
N = 2048
ROWS = 512
T = N // ROWS


def _chamfer_body(pt_ref, g_ref, loss_ref, zmin_ref):
    b = pl.program_id(0)
    t = pl.program_id(1)
    p = pt_ref[0]
    g = g_ref[0]
    d = ((p[:, 0:1] - g[0:1, :]) ** 2
         + (p[:, 1:2] - g[1:2, :]) ** 2
         + (p[:, 2:3] - g[2:3, :]) ** 2)

    colmin = jnp.min(d, axis=0, keepdims=True)
    zmin_new = jnp.where(t == 0, colmin,
                         jnp.minimum(zmin_ref[...], colmin))
    zmin_ref[...] = zmin_new

    partial = jnp.sum(jnp.sqrt(jnp.min(d, axis=1)))

    last_t = t == T - 1
    inc = partial + jnp.where(last_t, jnp.sum(jnp.sqrt(zmin_new)), 0.0)
    first = jnp.logical_and(b == 0, t == 0)
    acc = jnp.where(first, 0.0, loss_ref[0, 0]) + inc
    very_last = jnp.logical_and(b == B - 1, last_t)
    loss_ref[0, 0] = jnp.where(very_last, acc * (1.0 / (B * N)), acc)


def kernel(predict_pc, gt_pc):
    predict_t = jnp.transpose(predict_pc, (0, 2, 1))
    out = pl.pallas_call(
        _chamfer_body,
        grid=(B, T),
        in_specs=[
            pl.BlockSpec((1, ROWS, 3), lambda b, t: (b, t, 0)),
            pl.BlockSpec((1, 3, N), lambda b, t: (b, 0, 0)),
        ],
        out_specs=pl.BlockSpec((1, 1), lambda b, t: (0, 0),
                               memory_space=pltpu.SMEM),
        out_shape=jax.ShapeDtypeStruct((1, 1), jnp.float32),
        scratch_shapes=[pltpu.VMEM((1, N), jnp.float32)],
    )(predict_t, gt_pc)
    return out[0, 0]
